# baseline (speedup 1.0000x reference)
import functools

import jax
import jax.numpy as jnp
import numpy as np
from jax import lax
from jax.experimental import pallas as pl
from jax.experimental.pallas import tpu as pltpu
from jax.experimental.pallas import tpu_sc as plsc

_KEEP_EDGE = 0.5
_DIM_IN = 128
_DIM_OUT = 128
_EDGE_DIM = 16
_N_NODES = 10000
_N_EDGES = 320000


def _rotl32(x, r):
    return ((x << np.uint32(r)) | (x >> np.uint32(32 - r))).astype(np.uint32)


def _threefry2x32(k, x0, x1):
    k0, k1 = np.uint32(k[0]), np.uint32(k[1])
    ks = [k0, k1, np.uint32(k0 ^ k1 ^ np.uint32(0x1BD11BDA))]
    rot = [[13, 15, 26, 6], [17, 29, 16, 24]]
    x0 = (x0 + ks[0]).astype(np.uint32)
    x1 = (x1 + ks[1]).astype(np.uint32)
    for i in range(5):
        for r in rot[i % 2]:
            x0 = (x0 + x1).astype(np.uint32)
            x1 = _rotl32(x1, r)
            x1 = (x1 ^ x0).astype(np.uint32)
        x0 = (x0 + ks[(i + 1) % 3]).astype(np.uint32)
        x1 = (x1 + ks[(i + 2) % 3] + np.uint32(i + 1)).astype(np.uint32)
    return x0, x1


def _edge_mask(n, fold_data, thresh):
    fk0, fk1 = _threefry2x32(
        (np.uint32(0), np.uint32(0)), np.uint32([0]), np.uint32([fold_data])
    )
    o0, o1 = _threefry2x32(
        (fk0[0], fk1[0]), np.zeros(n, np.uint32), np.arange(n, dtype=np.uint32)
    )
    bits = (o0 ^ o1).astype(np.uint32)
    f = ((bits >> np.uint32(9)) | np.uint32(0x3F800000)).view(np.float32)
    return (f - np.float32(1.0)) < thresh


_MASK_NP = _edge_mask(_N_EDGES, 12345, _KEEP_EDGE)
_N_KEPT = int(_MASK_NP.sum())
_KEPT_NP = np.nonzero(_MASK_NP)[0].astype(np.int32)

_NW = 32
_K = 64
_NBUF = 4
_E_PAD = ((_N_KEPT + _NW * _K * _NBUF - 1) // (_NW * _K * _NBUF)) * (_NW * _K * _NBUF)
_NBLK = _E_PAD // _K
_CHUNKS = _NBLK // _NW
_N_ACC = 10112
_ROWS_PER_SUB = _N_ACC // 16

_PAD = _E_PAD - _N_KEPT
_OVER = 2 * _K
_KI_PAD_NP = np.concatenate(
    [_KEPT_NP, np.zeros(_PAD + _OVER, np.int32)]
)


def _h_matmul(x, w1, b2d):
    def body(x_ref, w_ref, b_ref, o_ref):
        o_ref[...] = (
            jnp.dot(x_ref[...], w_ref[...], preferred_element_type=jnp.float32)
            + b_ref[...]
        )

    return pl.pallas_call(
        body,
        grid=(10,),
        in_specs=[
            pl.BlockSpec((1000, 128), lambda i: (i, 0)),
            pl.BlockSpec((128, 128), lambda i: (0, 0)),
            pl.BlockSpec((1, 128), lambda i: (0, 0)),
        ],
        out_specs=pl.BlockSpec((1000, 128), lambda i: (i, 0)),
        out_shape=jax.ShapeDtypeStruct((_N_NODES, 128), jnp.float32),
    )(x, w1, b2d)


def _combine(hacc, eaacc, w2):
    def body(h_ref, ea_ref, w_ref, o_ref):
        ea = ea_ref[0] + ea_ref[1]
        o_ref[...] = (
            h_ref[0]
            + h_ref[1]
            + jnp.dot(ea, w_ref[...], preferred_element_type=jnp.float32)
        )

    return pl.pallas_call(
        body,
        grid=(10,),
        in_specs=[
            pl.BlockSpec((2, 1000, 128), lambda i: (0, i, 0)),
            pl.BlockSpec((2, 1000, 16), lambda i: (0, i, 0)),
            pl.BlockSpec((16, 128), lambda i: (0, 0)),
        ],
        out_specs=pl.BlockSpec((1000, 128), lambda i: (i, 0)),
        out_shape=jax.ShapeDtypeStruct((_N_NODES, 128), jnp.float32),
    )(hacc, eaacc, w2)


_NB = _NBUF


def _sc_aggregate(h, edge_attr, src_pad, dst_pad, ki, zh, zea):
    mesh = plsc.VectorSubcoreMesh(core_axis_name="c", subcore_axis_name="s")
    assert _CHUNKS % _NB == 0 and _NB == 4

    @functools.partial(
        pl.kernel,
        out_type=(
            jax.ShapeDtypeStruct((2, _N_ACC, 128), jnp.float32),
            jax.ShapeDtypeStruct((2, _N_ACC, 16), jnp.float32),
        ),
        mesh=mesh,
        scratch_types=(
            [pltpu.VMEM((_K,), jnp.int32) for _ in range(_NB)]
            + [pltpu.VMEM((_K,), jnp.int32) for _ in range(_NB)]
            + [pltpu.VMEM((_K,), jnp.int32) for _ in range(_NB)]
            + [pltpu.VMEM((_K, 128), jnp.float32) for _ in range(_NB)]
            + [pltpu.VMEM((_K, 16), jnp.float32) for _ in range(_NB)]
            + [
                pltpu.VMEM_SHARED((_N_ACC, 128), jnp.float32),
                pltpu.VMEM_SHARED((_N_ACC, 16), jnp.float32),
            ]
            + [pltpu.SemaphoreType.DMA for _ in range(3 * _NB)]
        ),
        compiler_params=pltpu.CompilerParams(use_tc_tiling_on_sc=False),
    )
    def k(h_hbm, ea_hbm, src_hbm, dst_hbm, ki_hbm, zh_hbm, zea_hbm,
          outh_hbm, outea_hbm, *scratch):
        kiv = scratch[0:_NB]
        srcv = scratch[_NB:2 * _NB]
        dstv = scratch[2 * _NB:3 * _NB]
        hrows = scratch[3 * _NB:4 * _NB]
        earows = scratch[4 * _NB:5 * _NB]
        hacc = scratch[5 * _NB]
        eaacc = scratch[5 * _NB + 1]
        sems = scratch[5 * _NB + 2:]
        isem = sems[0:_NB]
        gsem = sems[_NB:2 * _NB]
        ssem = sems[2 * _NB:]

        c = lax.axis_index("c")
        s = lax.axis_index("s")
        w = c * 16 + s
        rz = s * _ROWS_PER_SUB

        def issue_idx(b, blk):
            off = blk * _K
            pltpu.async_copy(src_hbm.at[pl.ds(off, _K)], srcv[b], isem[b])
            pltpu.async_copy(dst_hbm.at[pl.ds(off, _K)], dstv[b], isem[b])
            pltpu.async_copy(ki_hbm.at[pl.ds(off, _K)], kiv[b], isem[b])

        def wait_idx(b):
            pltpu.make_async_copy(src_hbm.at[pl.ds(0, _K)], srcv[b], isem[b]).wait()
            pltpu.make_async_copy(dst_hbm.at[pl.ds(0, _K)], dstv[b], isem[b]).wait()
            pltpu.make_async_copy(ki_hbm.at[pl.ds(0, _K)], kiv[b], isem[b]).wait()

        def issue_gathers(b):
            pltpu.async_copy(h_hbm.at[srcv[b]], hrows[b], gsem[b])
            pltpu.async_copy(ea_hbm.at[kiv[b]], earows[b], gsem[b])

        def wait_gathers(b):
            pltpu.make_async_copy(h_hbm.at[srcv[b]], hrows[b], gsem[b]).wait()
            pltpu.make_async_copy(ea_hbm.at[kiv[b]], earows[b], gsem[b]).wait()

        def issue_scatters(b):
            pltpu.async_copy(hrows[b], hacc.at[dstv[b]], ssem[b], add=True)
            pltpu.async_copy(earows[b], eaacc.at[dstv[b]], ssem[b], add=True)

        def wait_scatters(b):
            pltpu.make_async_copy(hrows[b], hacc.at[dstv[b]], ssem[b]).wait()
            pltpu.make_async_copy(earows[b], eaacc.at[dstv[b]], ssem[b]).wait()

        pltpu.sync_copy(
            zh_hbm.at[pl.ds(rz, _ROWS_PER_SUB)], hacc.at[pl.ds(rz, _ROWS_PER_SUB)]
        )
        pltpu.sync_copy(
            zea_hbm.at[pl.ds(rz, _ROWS_PER_SUB)], eaacc.at[pl.ds(rz, _ROWS_PER_SUB)]
        )
        plsc.subcore_barrier()

        base = w * _CHUNKS
        issue_idx(0, base)
        issue_idx(1, base + 1)
        wait_idx(0)
        issue_gathers(0)

        @pl.loop(0, _CHUNKS, step=_NB)
        def _(j0):
            for b in range(_NB):
                j = j0 + b
                b1 = (b + 1) % _NB
                b2 = (b + 2) % _NB
                wait_gathers(b)
                issue_scatters(b)

                @pl.when(j >= 2)
                def _():
                    wait_scatters(b2)

                issue_idx(b2, base + j + 2)
                wait_idx(b1)
                issue_gathers(b1)

        wait_scatters(2)
        wait_scatters(3)
        wait_gathers(0)
        wait_idx(1)
        plsc.subcore_barrier()
        pltpu.sync_copy(
            hacc.at[pl.ds(rz, _ROWS_PER_SUB)],
            outh_hbm.at[c, pl.ds(rz, _ROWS_PER_SUB)],
        )
        pltpu.sync_copy(
            eaacc.at[pl.ds(rz, _ROWS_PER_SUB)],
            outea_hbm.at[c, pl.ds(rz, _ROWS_PER_SUB)],
        )

    return k(h, edge_attr, src_pad, dst_pad, ki, zh, zea)


def kernel(x, edge_index, edge_attr, W, b):
    w1 = W[:_DIM_IN]
    w2 = W[_DIM_IN:]
    b2d = b.reshape(1, _DIM_OUT)

    h = _h_matmul(x, w1, b2d)

    kept = jnp.asarray(_KEPT_NP)
    ei = jnp.take(edge_index, kept, axis=1)

    dummy_dst = _N_NODES + (
        np.arange(_PAD + _OVER, dtype=np.int32) % (_N_ACC - _N_NODES)
    )
    src_pad = jnp.concatenate([ei[0], jnp.zeros((_PAD + _OVER,), jnp.int32)])
    dst_pad = jnp.concatenate([ei[1], jnp.asarray(dummy_dst)])
    ki = jnp.asarray(_KI_PAD_NP)

    zh = jnp.zeros((_N_ACC, 128), jnp.float32)
    zea = jnp.zeros((_N_ACC, 16), jnp.float32)

    hacc, eaacc = _sc_aggregate(h, edge_attr, src_pad, dst_pad, ki, zh, zea)
    out = _combine(hacc, eaacc, w2)
    return (out, ei, edge_attr)


# speedup vs baseline: 1.1247x; 1.1247x over previous
import functools

import jax
import jax.numpy as jnp
import numpy as np
from jax import lax
from jax.experimental import pallas as pl
from jax.experimental.pallas import tpu as pltpu
from jax.experimental.pallas import tpu_sc as plsc

_KEEP_EDGE = 0.5
_DIM_IN = 128
_DIM_OUT = 128
_EDGE_DIM = 16
_N_NODES = 10000
_N_EDGES = 320000


def _rotl32(x, r):
    return ((x << np.uint32(r)) | (x >> np.uint32(32 - r))).astype(np.uint32)


def _threefry2x32(k, x0, x1):
    k0, k1 = np.uint32(k[0]), np.uint32(k[1])
    ks = [k0, k1, np.uint32(k0 ^ k1 ^ np.uint32(0x1BD11BDA))]
    rot = [[13, 15, 26, 6], [17, 29, 16, 24]]
    x0 = (x0 + ks[0]).astype(np.uint32)
    x1 = (x1 + ks[1]).astype(np.uint32)
    for i in range(5):
        for r in rot[i % 2]:
            x0 = (x0 + x1).astype(np.uint32)
            x1 = _rotl32(x1, r)
            x1 = (x1 ^ x0).astype(np.uint32)
        x0 = (x0 + ks[(i + 1) % 3]).astype(np.uint32)
        x1 = (x1 + ks[(i + 2) % 3] + np.uint32(i + 1)).astype(np.uint32)
    return x0, x1


def _edge_mask(n, fold_data, thresh):
    fk0, fk1 = _threefry2x32(
        (np.uint32(0), np.uint32(0)), np.uint32([0]), np.uint32([fold_data])
    )
    o0, o1 = _threefry2x32(
        (fk0[0], fk1[0]), np.zeros(n, np.uint32), np.arange(n, dtype=np.uint32)
    )
    bits = (o0 ^ o1).astype(np.uint32)
    f = ((bits >> np.uint32(9)) | np.uint32(0x3F800000)).view(np.float32)
    return (f - np.float32(1.0)) < thresh


_MASK_NP = _edge_mask(_N_EDGES, 12345, _KEEP_EDGE)
_N_KEPT = int(_MASK_NP.sum())
_KEPT_NP = np.nonzero(_MASK_NP)[0].astype(np.int32)

_NW = 32
_K = 64
_NBUF = 4
_E_PAD = ((_N_KEPT + _NW * _K * _NBUF - 1) // (_NW * _K * _NBUF)) * (_NW * _K * _NBUF)
_C0 = 136
_C1 = 24
_NBLK = _E_PAD // _K
_CHUNKS = _NBLK // _NW
_N_ACC = 10112
_ROWS_PER_SUB = _N_ACC // 16

_PAD = _E_PAD - _N_KEPT
_OVER = 2 * _K
_KI_PAD_NP = np.concatenate(
    [_KEPT_NP, np.zeros(_PAD + _OVER, np.int32)]
)


def _h_matmul(x, w1, b2d):
    def body(x_ref, w_ref, b_ref, o_ref):
        o_ref[...] = (
            jnp.dot(x_ref[...], w_ref[...], preferred_element_type=jnp.float32)
            + b_ref[...]
        )

    return pl.pallas_call(
        body,
        grid=(10,),
        in_specs=[
            pl.BlockSpec((1000, 128), lambda i: (i, 0)),
            pl.BlockSpec((128, 128), lambda i: (0, 0)),
            pl.BlockSpec((1, 128), lambda i: (0, 0)),
        ],
        out_specs=pl.BlockSpec((1000, 128), lambda i: (i, 0)),
        out_shape=jax.ShapeDtypeStruct((_N_NODES, 128), jnp.float32),
    )(x, w1, b2d)


def _combine(hacc, eaacc, w2):
    def body(h_ref, ea_ref, w_ref, o_ref):
        ea = ea_ref[0] + ea_ref[1]
        o_ref[...] = (
            h_ref[0]
            + h_ref[1]
            + jnp.dot(ea, w_ref[...], preferred_element_type=jnp.float32)
        )

    return pl.pallas_call(
        body,
        grid=(10,),
        in_specs=[
            pl.BlockSpec((2, 1000, 128), lambda i: (0, i, 0)),
            pl.BlockSpec((2, 1000, 16), lambda i: (0, i, 0)),
            pl.BlockSpec((16, 128), lambda i: (0, 0)),
        ],
        out_specs=pl.BlockSpec((1000, 128), lambda i: (i, 0)),
        out_shape=jax.ShapeDtypeStruct((_N_NODES, 128), jnp.float32),
    )(hacc, eaacc, w2)


_NB = _NBUF


def _sc_aggregate(h, edge_attr, src_pad, dst_pad, ki, zh, zea):
    mesh = plsc.VectorSubcoreMesh(core_axis_name="c", subcore_axis_name="s")
    assert _CHUNKS % _NB == 0 and _NB == 4
    assert _C0 + _C1 == 2 * _CHUNKS and _C0 % _NB == 0 and _C1 % _NB == 0

    @functools.partial(
        pl.kernel,
        out_type=(
            jax.ShapeDtypeStruct((2, _N_ACC, 128), jnp.float32),
            jax.ShapeDtypeStruct((2, _N_ACC, 16), jnp.float32),
        ),
        mesh=mesh,
        scratch_types=(
            [pltpu.VMEM((_K,), jnp.int32) for _ in range(_NB)]
            + [pltpu.VMEM((_K,), jnp.int32) for _ in range(_NB)]
            + [pltpu.VMEM((_K,), jnp.int32) for _ in range(_NB)]
            + [pltpu.VMEM((_K, 128), jnp.float32) for _ in range(_NB)]
            + [pltpu.VMEM((_K, 16), jnp.float32) for _ in range(_NB)]
            + [
                pltpu.VMEM_SHARED((_N_ACC, 128), jnp.float32),
                pltpu.VMEM_SHARED((_N_ACC, 16), jnp.float32),
            ]
            + [pltpu.SemaphoreType.DMA for _ in range(3 * _NB)]
        ),
        compiler_params=pltpu.CompilerParams(use_tc_tiling_on_sc=False),
    )
    def k(h_hbm, ea_hbm, src_hbm, dst_hbm, ki_hbm, zh_hbm, zea_hbm,
          outh_hbm, outea_hbm, *scratch):
        kiv = scratch[0:_NB]
        srcv = scratch[_NB:2 * _NB]
        dstv = scratch[2 * _NB:3 * _NB]
        hrows = scratch[3 * _NB:4 * _NB]
        earows = scratch[4 * _NB:5 * _NB]
        hacc = scratch[5 * _NB]
        eaacc = scratch[5 * _NB + 1]
        sems = scratch[5 * _NB + 2:]
        isem = sems[0:_NB]
        gsem = sems[_NB:2 * _NB]
        ssem = sems[2 * _NB:]

        c = lax.axis_index("c")
        s = lax.axis_index("s")
        rz = s * _ROWS_PER_SUB
        n_chunks = jnp.where(c == 0, _C0, _C1)
        base = jnp.where(c == 0, s * _C0, 16 * _C0 + s * _C1)

        def issue_idx(b, blk):
            off = blk * _K
            pltpu.async_copy(src_hbm.at[pl.ds(off, _K)], srcv[b], isem[b])
            pltpu.async_copy(dst_hbm.at[pl.ds(off, _K)], dstv[b], isem[b])
            pltpu.async_copy(ki_hbm.at[pl.ds(off, _K)], kiv[b], isem[b])

        def wait_idx(b):
            pltpu.make_async_copy(src_hbm.at[pl.ds(0, _K)], srcv[b], isem[b]).wait()
            pltpu.make_async_copy(dst_hbm.at[pl.ds(0, _K)], dstv[b], isem[b]).wait()
            pltpu.make_async_copy(ki_hbm.at[pl.ds(0, _K)], kiv[b], isem[b]).wait()

        def issue_gathers(b):
            pltpu.async_copy(h_hbm.at[srcv[b]], hrows[b], gsem[b])
            pltpu.async_copy(ea_hbm.at[kiv[b]], earows[b], gsem[b])

        def wait_gathers(b):
            pltpu.make_async_copy(h_hbm.at[srcv[b]], hrows[b], gsem[b]).wait()
            pltpu.make_async_copy(ea_hbm.at[kiv[b]], earows[b], gsem[b]).wait()

        def issue_scatters(b):
            pltpu.async_copy(hrows[b], hacc.at[dstv[b]], ssem[b], add=True)
            pltpu.async_copy(earows[b], eaacc.at[dstv[b]], ssem[b], add=True)

        def wait_scatters(b):
            pltpu.make_async_copy(hrows[b], hacc.at[dstv[b]], ssem[b]).wait()
            pltpu.make_async_copy(earows[b], eaacc.at[dstv[b]], ssem[b]).wait()

        pltpu.sync_copy(
            zh_hbm.at[pl.ds(rz, _ROWS_PER_SUB)], hacc.at[pl.ds(rz, _ROWS_PER_SUB)]
        )
        pltpu.sync_copy(
            zea_hbm.at[pl.ds(rz, _ROWS_PER_SUB)], eaacc.at[pl.ds(rz, _ROWS_PER_SUB)]
        )
        plsc.subcore_barrier()

        issue_idx(0, base)
        issue_idx(1, base + 1)
        wait_idx(0)
        issue_gathers(0)

        @pl.loop(0, n_chunks, step=_NB)
        def _(j0):
            for b in range(_NB):
                j = j0 + b
                b1 = (b + 1) % _NB
                b2 = (b + 2) % _NB
                wait_gathers(b)
                issue_scatters(b)

                @pl.when(j >= 2)
                def _():
                    wait_scatters(b2)

                issue_idx(b2, base + j + 2)
                wait_idx(b1)
                issue_gathers(b1)

        wait_scatters(2)
        wait_scatters(3)
        wait_gathers(0)
        wait_idx(1)
        plsc.subcore_barrier()
        pltpu.sync_copy(
            hacc.at[pl.ds(rz, _ROWS_PER_SUB)],
            outh_hbm.at[c, pl.ds(rz, _ROWS_PER_SUB)],
        )
        pltpu.sync_copy(
            eaacc.at[pl.ds(rz, _ROWS_PER_SUB)],
            outea_hbm.at[c, pl.ds(rz, _ROWS_PER_SUB)],
        )

    return k(h, edge_attr, src_pad, dst_pad, ki, zh, zea)


def kernel(x, edge_index, edge_attr, W, b):
    w1 = W[:_DIM_IN]
    w2 = W[_DIM_IN:]
    b2d = b.reshape(1, _DIM_OUT)

    h = _h_matmul(x, w1, b2d)

    kept = jnp.asarray(_KEPT_NP)
    ei = jnp.take(edge_index, kept, axis=1)

    dummy_dst = _N_NODES + (
        np.arange(_PAD + _OVER, dtype=np.int32) % (_N_ACC - _N_NODES)
    )
    src_pad = jnp.concatenate([ei[0], jnp.zeros((_PAD + _OVER,), jnp.int32)])
    dst_pad = jnp.concatenate([ei[1], jnp.asarray(dummy_dst)])
    ki = jnp.asarray(_KI_PAD_NP)

    zh = jnp.zeros((_N_ACC, 128), jnp.float32)
    zea = jnp.zeros((_N_ACC, 16), jnp.float32)

    hacc, eaacc = _sc_aggregate(h, edge_attr, src_pad, dst_pad, ki, zh, zea)
    out = _combine(hacc, eaacc, w2)
    return (out, ei, edge_attr)
